# baseline (speedup 1.0000x reference)
import functools

import jax
import jax.numpy as jnp
from jax import lax
from jax.experimental import pallas as pl
from jax.experimental.pallas import tpu as pltpu

DT = 0.02
TEMPERATURE = 1.0


def _router_block(x_ref, ag_ref, u_ref, wb_ref, ub_ref, g_ref, gb_ref,
                  fo_ref, usage_ref,
                  *, n_blocks, b_total, n_chunks):
    i = pl.program_id(0)
    E = g_ref.shape[0]
    Bb = x_ref.shape[0]
    Cb = Bb // n_chunks

    def _chunk(c):
        sl = pl.ds(c * Cb, Cb)
        z = lax.dot_general(x_ref[sl, :], u_ref[...],
                            (((1,), (1,)), ((), ())),
                            preferred_element_type=jnp.float32)
        z = (wb_ref[...] + z) + ub_ref[...]
        h = DT * jnp.tanh(z)

        logits = lax.dot_general(h, g_ref[...],
                                 (((1,), (1,)), ((), ())),
                                 preferred_element_type=jnp.float32)
        logits = logits + gb_ref[...]
        temp = jnp.clip(TEMPERATURE / (ag_ref[sl, :] + 1e-06), 0.1, 5.0)
        logits = logits / temp

        m = jnp.max(logits, axis=-1, keepdims=True)
        e = jnp.exp(logits - m)
        probs = e / jnp.sum(e, axis=-1, keepdims=True)

        idx = lax.broadcasted_iota(jnp.int32, (Cb, E), 1)
        m1 = jnp.max(probs, axis=-1, keepdims=True)
        i1 = jnp.min(jnp.where(probs == m1, idx, E), axis=-1, keepdims=True)
        masked = jnp.where(idx == i1, -jnp.inf, probs)
        m2 = jnp.max(masked, axis=-1, keepdims=True)
        i2 = jnp.min(jnp.where(masked == m2, idx, E), axis=-1, keepdims=True)

        denom = (m1 + m2) + 1e-08
        fo_ref[sl, :] = jnp.concatenate(
            [probs, m1 / denom, m2 / denom,
             i1.astype(jnp.float32), i2.astype(jnp.float32)], axis=1)

        return jnp.sum((idx == i1).astype(jnp.float32)
                       + (idx == i2).astype(jnp.float32),
                       axis=0, keepdims=True)

    cnt = _chunk(0)
    for c in range(1, n_chunks):
        cnt = cnt + _chunk(c)

    @pl.when(i == 0)
    def _init():
        usage_ref[...] = jnp.zeros_like(usage_ref)

    usage_ref[...] += cnt

    @pl.when(i == n_blocks - 1)
    def _finish():
        usage_ref[...] = 0.01 * (usage_ref[...] / float(b_total))


def kernel(x, attn_gain, W_w, W_b, U_w, U_b, V_w, V_b, G_w, G_b):
    B, D = x.shape
    H = U_w.shape[0]
    E = G_w.shape[0]
    Bb = min(2048, B)
    n_blocks = B // Bb

    wb2 = W_b.reshape(1, H)
    ub2 = U_b.reshape(1, H)
    gb2 = G_b.reshape(1, E)

    grid = (n_blocks,)
    out_shape = (
        jax.ShapeDtypeStruct((B, E + 4), jnp.float32),
        jax.ShapeDtypeStruct((1, E), jnp.float32),
    )
    in_specs = [
        pl.BlockSpec((Bb, D), lambda i: (i, 0)),
        pl.BlockSpec((Bb, 1), lambda i: (i, 0)),
        pl.BlockSpec((H, D), lambda i: (0, 0)),
        pl.BlockSpec((1, H), lambda i: (0, 0)),
        pl.BlockSpec((1, H), lambda i: (0, 0)),
        pl.BlockSpec((E, H), lambda i: (0, 0)),
        pl.BlockSpec((1, E), lambda i: (0, 0)),
    ]
    out_specs = (
        pl.BlockSpec((Bb, E + 4), lambda i: (i, 0)),
        pl.BlockSpec((1, E), lambda i: (0, 0)),
    )

    fo, usage = pl.pallas_call(
        functools.partial(_router_block, n_blocks=n_blocks, b_total=B,
                          n_chunks=8),
        grid=grid,
        in_specs=in_specs,
        out_specs=out_specs,
        out_shape=out_shape,
        compiler_params=pltpu.CompilerParams(
            dimension_semantics=("arbitrary",),
        ),
    )(x, attn_gain, U_w, wb2, ub2, G_w, gb2)

    probs = fo[:, :E]
    tw = fo[:, E:E + 2]
    ti = fo[:, E + 2:E + 4].astype(jnp.int32)
    return tw, ti, probs, usage.reshape(E)


# speedup vs baseline: 1.0592x; 1.0592x over previous
import functools

import jax
import jax.numpy as jnp
from jax import lax
from jax.experimental import pallas as pl
from jax.experimental.pallas import tpu as pltpu

DT = 0.02
TEMPERATURE = 1.0


def _router_block(x_ref, ag_ref, u_ref, wb_ref, ub_ref, g_ref, gb_ref,
                  tw_ref, ti_ref, probs_ref, usage_ref,
                  *, n_blocks, b_total, n_chunks):
    i = pl.program_id(0)
    E = g_ref.shape[0]
    Bb = x_ref.shape[0]
    Cb = Bb // n_chunks

    def _chunk(c):
        sl = pl.ds(c * Cb, Cb)
        z = lax.dot_general(x_ref[sl, :], u_ref[...],
                            (((1,), (1,)), ((), ())),
                            preferred_element_type=jnp.float32)
        z = (wb_ref[...] + z) + ub_ref[...]
        h = DT * jnp.tanh(z)

        logits = lax.dot_general(h, g_ref[...],
                                 (((1,), (1,)), ((), ())),
                                 preferred_element_type=jnp.float32)
        logits = logits + gb_ref[...]
        temp = jnp.clip(TEMPERATURE / (ag_ref[sl, :] + 1e-06), 0.1, 5.0)
        logits = logits / temp

        m = jnp.max(logits, axis=-1, keepdims=True)
        e = jnp.exp(logits - m)
        probs = e / jnp.sum(e, axis=-1, keepdims=True)
        probs_ref[sl, :] = probs

        idx = lax.broadcasted_iota(jnp.int32, (Cb, E), 1)
        m1 = jnp.max(probs, axis=-1, keepdims=True)
        i1 = jnp.min(jnp.where(probs == m1, idx, E), axis=-1, keepdims=True)
        masked = jnp.where(idx == i1, -jnp.inf, probs)
        m2 = jnp.max(masked, axis=-1, keepdims=True)
        i2 = jnp.min(jnp.where(masked == m2, idx, E), axis=-1, keepdims=True)

        denom = (m1 + m2) + 1e-08
        tw_ref[sl, :] = jnp.concatenate([m1 / denom, m2 / denom], axis=1)
        ti_ref[sl, :] = jnp.concatenate([i1, i2], axis=1)

        return jnp.sum((idx == i1).astype(jnp.float32)
                       + (idx == i2).astype(jnp.float32),
                       axis=0, keepdims=True)

    cnt = _chunk(0)
    for c in range(1, n_chunks):
        cnt = cnt + _chunk(c)

    @pl.when(i == 0)
    def _init():
        usage_ref[...] = jnp.zeros_like(usage_ref)

    usage_ref[...] += cnt

    @pl.when(i == n_blocks - 1)
    def _finish():
        usage_ref[...] = 0.01 * (usage_ref[...] / float(b_total))


def kernel(x, attn_gain, W_w, W_b, U_w, U_b, V_w, V_b, G_w, G_b):
    B, D = x.shape
    H = U_w.shape[0]
    E = G_w.shape[0]
    Bb = min(2048, B)
    n_blocks = B // Bb

    wb2 = W_b.reshape(1, H)
    ub2 = U_b.reshape(1, H)
    gb2 = G_b.reshape(1, E)

    grid = (n_blocks,)
    out_shape = (
        jax.ShapeDtypeStruct((B, 2), jnp.float32),
        jax.ShapeDtypeStruct((B, 2), jnp.int32),
        jax.ShapeDtypeStruct((B, E), jnp.float32),
        jax.ShapeDtypeStruct((1, E), jnp.float32),
    )
    in_specs = [
        pl.BlockSpec((Bb, D), lambda i: (i, 0)),
        pl.BlockSpec((Bb, 1), lambda i: (i, 0)),
        pl.BlockSpec((H, D), lambda i: (0, 0)),
        pl.BlockSpec((1, H), lambda i: (0, 0)),
        pl.BlockSpec((1, H), lambda i: (0, 0)),
        pl.BlockSpec((E, H), lambda i: (0, 0)),
        pl.BlockSpec((1, E), lambda i: (0, 0)),
    ]
    out_specs = (
        pl.BlockSpec((Bb, 2), lambda i: (i, 0)),
        pl.BlockSpec((Bb, 2), lambda i: (i, 0)),
        pl.BlockSpec((Bb, E), lambda i: (i, 0)),
        pl.BlockSpec((1, E), lambda i: (0, 0)),
    )

    tw, ti, probs, usage = pl.pallas_call(
        functools.partial(_router_block, n_blocks=n_blocks, b_total=B,
                          n_chunks=8),
        grid=grid,
        in_specs=in_specs,
        out_specs=out_specs,
        out_shape=out_shape,
        compiler_params=pltpu.CompilerParams(
            dimension_semantics=("arbitrary",),
        ),
    )(x, attn_gain, U_w, wb2, ub2, G_w, gb2)

    return tw, ti, probs, usage.reshape(E)


# speedup vs baseline: 1.0899x; 1.0290x over previous
import functools

import jax
import jax.numpy as jnp
from jax import lax
from jax.experimental import pallas as pl
from jax.experimental.pallas import tpu as pltpu

DT = 0.02
TEMPERATURE = 1.0


def _router_block(x_ref, ag_ref, u_ref, wb_ref, ub_ref, g_ref, gb_ref,
                  tw_ref, ti_ref, probs_ref, usage_ref,
                  *, n_blocks, b_total, n_chunks):
    i = pl.program_id(0)
    E = g_ref.shape[0]
    Bb = x_ref.shape[0]
    Cb = Bb // n_chunks

    temp_col = jnp.transpose(
        jnp.clip(TEMPERATURE / (ag_ref[...] + 1e-06), 0.1, 5.0))

    def _chunk(c):
        sl = pl.ds(c * Cb, Cb)
        z = lax.dot_general(x_ref[sl, :], u_ref[...],
                            (((1,), (1,)), ((), ())),
                            preferred_element_type=jnp.float32)
        z = (wb_ref[...] + z) + ub_ref[...]
        h = DT * jnp.tanh(z)

        logits = lax.dot_general(h, g_ref[...],
                                 (((1,), (1,)), ((), ())),
                                 preferred_element_type=jnp.float32)
        logits = logits + gb_ref[...]
        logits = logits / lax.slice_in_dim(temp_col, c * Cb, (c + 1) * Cb, axis=0)

        m = jnp.max(logits, axis=-1, keepdims=True)
        e = jnp.exp(logits - m)
        probs = e / jnp.sum(e, axis=-1, keepdims=True)
        probs_ref[sl, :] = probs

        idx = lax.broadcasted_iota(jnp.int32, (Cb, E), 1)
        m1 = jnp.max(probs, axis=-1, keepdims=True)
        i1 = jnp.min(jnp.where(probs == m1, idx, E), axis=-1, keepdims=True)
        masked = jnp.where(idx == i1, -jnp.inf, probs)
        m2 = jnp.max(masked, axis=-1, keepdims=True)
        i2 = jnp.min(jnp.where(masked == m2, idx, E), axis=-1, keepdims=True)

        denom = (m1 + m2) + 1e-08
        tw_ref[sl, :] = jnp.concatenate([m1 / denom, m2 / denom], axis=1)
        ti_ref[sl, :] = jnp.concatenate([i1, i2], axis=1)

        return jnp.sum((idx == i1).astype(jnp.float32)
                       + (idx == i2).astype(jnp.float32),
                       axis=0, keepdims=True)

    cnt = _chunk(0)
    for c in range(1, n_chunks):
        cnt = cnt + _chunk(c)

    @pl.when(i == 0)
    def _init():
        usage_ref[...] = jnp.zeros_like(usage_ref)

    usage_ref[...] += cnt

    @pl.when(i == n_blocks - 1)
    def _finish():
        usage_ref[...] = 0.01 * (usage_ref[...] / float(b_total))


def kernel(x, attn_gain, W_w, W_b, U_w, U_b, V_w, V_b, G_w, G_b):
    B, D = x.shape
    H = U_w.shape[0]
    E = G_w.shape[0]
    Bb = min(2048, B)
    n_blocks = B // Bb

    wb2 = W_b.reshape(1, H)
    ub2 = U_b.reshape(1, H)
    gb2 = G_b.reshape(1, E)

    grid = (n_blocks,)
    out_shape = (
        jax.ShapeDtypeStruct((B, 2), jnp.float32),
        jax.ShapeDtypeStruct((B, 2), jnp.int32),
        jax.ShapeDtypeStruct((B, E), jnp.float32),
        jax.ShapeDtypeStruct((1, E), jnp.float32),
    )
    in_specs = [
        pl.BlockSpec((Bb, D), lambda i: (i, 0)),
        pl.BlockSpec((1, Bb), lambda i: (0, i)),
        pl.BlockSpec((H, D), lambda i: (0, 0)),
        pl.BlockSpec((1, H), lambda i: (0, 0)),
        pl.BlockSpec((1, H), lambda i: (0, 0)),
        pl.BlockSpec((E, H), lambda i: (0, 0)),
        pl.BlockSpec((1, E), lambda i: (0, 0)),
    ]
    out_specs = (
        pl.BlockSpec((Bb, 2), lambda i: (i, 0)),
        pl.BlockSpec((Bb, 2), lambda i: (i, 0)),
        pl.BlockSpec((Bb, E), lambda i: (i, 0)),
        pl.BlockSpec((1, E), lambda i: (0, 0)),
    )

    tw, ti, probs, usage = pl.pallas_call(
        functools.partial(_router_block, n_blocks=n_blocks, b_total=B,
                          n_chunks=8),
        grid=grid,
        in_specs=in_specs,
        out_specs=out_specs,
        out_shape=out_shape,
        compiler_params=pltpu.CompilerParams(
            dimension_semantics=("arbitrary",),
        ),
    )(x, attn_gain.reshape(1, B), U_w, wb2, ub2, G_w, gb2)

    return tw, ti, probs, usage.reshape(E)


# speedup vs baseline: 1.0918x; 1.0017x over previous
import functools

import jax
import jax.numpy as jnp
from jax import lax
from jax.experimental import pallas as pl
from jax.experimental.pallas import tpu as pltpu

DT = 0.02
TEMPERATURE = 1.0


def _router_block(x_ref, ag_ref, u_ref, wb_ref, ub_ref, g_ref, gb_ref,
                  tw_ref, ti_ref, probs_ref, usage_ref,
                  *, n_blocks, b_total, n_chunks):
    i = pl.program_id(0)
    E = g_ref.shape[0]
    Bb = x_ref.shape[0]
    Cb = Bb // n_chunks

    temp_col = jnp.transpose(
        jnp.clip(TEMPERATURE / (ag_ref[...] + 1e-06), 0.1, 5.0))

    def _chunk(c):
        sl = pl.ds(c * Cb, Cb)
        z = lax.dot_general(x_ref[sl, :], u_ref[...],
                            (((1,), (1,)), ((), ())),
                            preferred_element_type=jnp.float32)
        z = (wb_ref[...] + z) + ub_ref[...]
        h = DT * jnp.tanh(z)

        logits = lax.dot_general(h, g_ref[...],
                                 (((1,), (1,)), ((), ())),
                                 preferred_element_type=jnp.float32)
        logits = logits + gb_ref[...]
        logits = logits / lax.slice_in_dim(temp_col, c * Cb, (c + 1) * Cb, axis=0)

        m = jnp.max(logits, axis=-1, keepdims=True)
        e = jnp.exp(logits - m)
        probs = e / jnp.sum(e, axis=-1, keepdims=True)
        probs_ref[sl, :] = probs

        idx = lax.broadcasted_iota(jnp.int32, (Cb, E), 1)
        m1 = jnp.max(probs, axis=-1, keepdims=True)
        i1 = jnp.min(jnp.where(probs == m1, idx, E), axis=-1, keepdims=True)
        masked = jnp.where(idx == i1, -jnp.inf, probs)
        m2 = jnp.max(masked, axis=-1, keepdims=True)
        i2 = jnp.min(jnp.where(masked == m2, idx, E), axis=-1, keepdims=True)

        denom = (m1 + m2) + 1e-08
        tw_ref[sl, :] = jnp.concatenate([m1 / denom, m2 / denom], axis=1)
        ti_ref[sl, :] = jnp.concatenate([i1, i2], axis=1)

        return jnp.sum((idx == i1).astype(jnp.float32)
                       + (idx == i2).astype(jnp.float32),
                       axis=0, keepdims=True)

    cnt = _chunk(0)
    for c in range(1, n_chunks):
        cnt = cnt + _chunk(c)

    @pl.when(i == 0)
    def _init():
        usage_ref[...] = jnp.zeros_like(usage_ref)

    usage_ref[...] += cnt

    @pl.when(i == n_blocks - 1)
    def _finish():
        usage_ref[...] = 0.01 * (usage_ref[...] / float(b_total))


def kernel(x, attn_gain, W_w, W_b, U_w, U_b, V_w, V_b, G_w, G_b):
    B, D = x.shape
    H = U_w.shape[0]
    E = G_w.shape[0]
    Bb = min(2048, B)
    n_blocks = B // Bb

    wb2 = W_b.reshape(1, H)
    ub2 = U_b.reshape(1, H)
    gb2 = G_b.reshape(1, E)

    grid = (n_blocks,)
    out_shape = (
        jax.ShapeDtypeStruct((B, 2), jnp.float32),
        jax.ShapeDtypeStruct((B, 2), jnp.int32),
        jax.ShapeDtypeStruct((B, E), jnp.float32),
        jax.ShapeDtypeStruct((1, E), jnp.float32),
    )
    in_specs = [
        pl.BlockSpec((Bb, D), lambda i: (i, 0)),
        pl.BlockSpec((1, Bb), lambda i: (0, i)),
        pl.BlockSpec((H, D), lambda i: (0, 0)),
        pl.BlockSpec((1, H), lambda i: (0, 0)),
        pl.BlockSpec((1, H), lambda i: (0, 0)),
        pl.BlockSpec((E, H), lambda i: (0, 0)),
        pl.BlockSpec((1, E), lambda i: (0, 0)),
    ]
    out_specs = (
        pl.BlockSpec((Bb, 2), lambda i: (i, 0)),
        pl.BlockSpec((Bb, 2), lambda i: (i, 0)),
        pl.BlockSpec((Bb, E), lambda i: (i, 0)),
        pl.BlockSpec((1, E), lambda i: (0, 0)),
    )

    tw, ti, probs, usage = pl.pallas_call(
        functools.partial(_router_block, n_blocks=n_blocks, b_total=B,
                          n_chunks=4),
        grid=grid,
        in_specs=in_specs,
        out_specs=out_specs,
        out_shape=out_shape,
        compiler_params=pltpu.CompilerParams(
            dimension_semantics=("arbitrary",),
        ),
    )(x, attn_gain.reshape(1, B), U_w, wb2, ub2, G_w, gb2)

    return tw, ti, probs, usage.reshape(E)
